# Initial kernel scaffold; baseline (speedup 1.0000x reference)
import jax
import jax.numpy as jnp
from jax.experimental import pallas as pl


def kernel(x, edge_attr, params, edge_index, batch):
    raise NotImplementedError("write your pallas kernel here")


# speedup vs baseline: 9.9973x; 9.9973x over previous
import functools

import jax
import jax.numpy as jnp
from jax.experimental import pallas as pl

N = 10000
E = 320000
D = 128
DE = 16
H = 4
C = 32
B = 64

NPAD = 10240


def _layer_norm(x, g, b, eps=1e-5):
    mu = jnp.mean(x, axis=-1, keepdims=True)
    var = jnp.mean((x - mu) ** 2, axis=-1, keepdims=True)
    return (x - mu) / jnp.sqrt(var + eps) * g + b


def _matmul_relu_kernel(x_ref, w_ref, b_ref, o_ref):
    o_ref[...] = jax.nn.relu(
        jnp.dot(x_ref[...], w_ref[...], preferred_element_type=jnp.float32)
        + b_ref[...]
    )


def _proj_relu(x, w, b):
    m = x.shape[0]
    bm = 256
    grid = (m // bm,)
    return pl.pallas_call(
        _matmul_relu_kernel,
        grid=grid,
        in_specs=[
            pl.BlockSpec((bm, x.shape[1]), lambda i: (i, 0)),
            pl.BlockSpec((x.shape[1], w.shape[1]), lambda i: (0, 0)),
            pl.BlockSpec((1, w.shape[1]), lambda i: (0, 0)),
        ],
        out_specs=pl.BlockSpec((bm, w.shape[1]), lambda i: (i, 0)),
        out_shape=jax.ShapeDtypeStruct((m, w.shape[1]), jnp.float32),
    )(x, w, b[None])


def _gat_layer(x, h, edge_attr, edge_index, g):
    n = x.shape[0]
    inp = jnp.concatenate([x, h], axis=-1)
    xw = inp @ g['W']
    xwr = xw.reshape(n, H, C)
    a_src = jnp.sum(xwr * g['att_src'][None], axis=-1)
    a_dst = jnp.sum(xwr * g['att_dst'][None], axis=-1)
    A = (g['W_e'].reshape(DE, H, C) * g['att_e'][None]).sum(-1)
    e4 = edge_attr @ A
    src, dst = edge_index[0], edge_index[1]

    alpha = jax.nn.leaky_relu(a_src[src] + a_dst[dst] + e4, 0.2)
    p = jnp.exp(alpha)
    num = jax.ops.segment_sum(xw[src] * jnp.repeat(p, C, axis=1), dst, n)
    den = jax.ops.segment_sum(p, dst, n)
    se4 = jax.ops.segment_sum(e4, dst, n)
    deg = jax.ops.segment_sum(jnp.ones(E, jnp.float32), dst, n)

    loop_a = se4 / jnp.maximum(deg, 1.0)[:, None]
    p_self = jnp.exp(jax.nn.leaky_relu(a_src + a_dst + loop_a, 0.2))
    den_t = den + p_self + 1e-16
    num_t = num + xw * jnp.repeat(p_self, C, axis=1)
    out = num_t / jnp.repeat(den_t, C, axis=1)
    return out + g['bias']


def kernel(x, edge_attr, params, edge_index, batch):
    p = params
    h = _proj_relu(jnp.pad(x, ((0, NPAD - N), (0, 0))), p['W_in'], p['b_in'])[:N]
    for l in range(2):
        g = p['gat'][l]
        h = _gat_layer(x, h, edge_attr, edge_index, g)
        h = jax.nn.elu(_layer_norm(h, g['ln_g'], g['ln_b']))
    ge = _layer_norm(jax.nn.relu(h @ p['Wg'] + p['bg']), p['lng_g'], p['lng_b'])
    cnt = jax.ops.segment_sum(jnp.ones(N, jnp.float32), batch, B)
    ge = jax.ops.segment_sum(ge, batch, B) / jnp.maximum(cnt, 1.0)[:, None]
    h = _layer_norm(jax.nn.relu(h @ p['Wo'] + p['bo']), p['lnl_g'], p['lnl_b'])
    node_in = jnp.concatenate([x, h], axis=-1)
    z = jax.nn.relu(node_in @ p['Wn1'] + p['bn1'])
    z = jax.nn.relu(z @ p['Wn2'] + p['bn2'])
    no = (z @ p['Wn3'] + p['bn3'])[:, 0]
    z = jax.nn.relu(ge @ p['Wg1'] + p['bg1'])
    z = jax.nn.relu(z @ p['Wg2'] + p['bg2'])
    go = (z @ p['Wg3'] + p['bg3'])[:, 0]
    return jnp.concatenate([no, go], axis=-1)


# speedup vs baseline: 52.1333x; 5.2147x over previous
import functools

import jax
import jax.numpy as jnp
from jax import lax
from jax.experimental import pallas as pl
from jax.experimental.pallas import tpu as pltpu
from jax.experimental.pallas import tpu_sc as plsc

N = 10000
E = 320000
D = 128
DE = 16
H = 4
C = 32
B = 64

NC = 2
NS = 16
NW = NC * NS
EW = E // NW
CH = 128
NCH = EW // CH
TAIL = EW - NCH * CH
NR = 144
ZTILE = 632
ZLAST = N - 15 * ZTILE


def _sc_edge_pass(table, adst, src, dst, e4):
    mesh = plsc.VectorSubcoreMesh(core_axis_name="c", subcore_axis_name="s")

    @functools.partial(
        pl.kernel,
        out_type=jax.ShapeDtypeStruct((NC, N, NR), jnp.float32),
        mesh=mesh,
        scratch_types=[
            pltpu.VMEM_SHARED((N, NR), jnp.float32),
            pltpu.VMEM((CH, NR), jnp.float32),
            pltpu.VMEM((CH,), jnp.int32),
            pltpu.VMEM((CH,), jnp.int32),
            pltpu.VMEM((CH, 16), jnp.float32),
            pltpu.VMEM((CH, 4), jnp.float32),
            pltpu.VMEM((TAIL,), jnp.int32),
            pltpu.VMEM((TAIL,), jnp.int32),
            pltpu.VMEM((8, NR), jnp.float32),
            pltpu.SemaphoreType.DMA,
            pltpu.SemaphoreType.DMA,
        ],
        compiler_params=pltpu.CompilerParams(use_tc_tiling_on_sc=False,
                                             needs_layout_passes=False),
    )
    def body(table_h, adst_h, src_h, dst_h, e4_h, out_h,
             acc, rows, srci, dsti, adr, e4v, srct, dstt, zbuf,
             sem1, sem2):
        cid = lax.axis_index("c")
        sid = lax.axis_index("s")
        wid = sid * NC + cid
        lane = lax.broadcasted_iota(jnp.int32, (16,), 0)
        el = lane >> 2
        hh = lane & 3
        zv = jnp.zeros((16,), jnp.float32)

        for r in range(8):
            for cgrp in range(NR // 16):
                zbuf[r, pl.ds(cgrp * 16, 16)] = zv
        tb = sid * ZTILE
        nz = jnp.where(sid == NS - 1, ZLAST // 8, ZTILE // 8)

        def zstep(k, _):
            off = pl.multiple_of(tb + k * 8, 8)
            pltpu.sync_copy(zbuf, acc.at[pl.ds(off, 8)])
            return 0

        lax.fori_loop(0, nz, zstep, 0)
        plsc.subcore_barrier()

        def compute(nch4):
            def pstep(j, _):
                rowv = j * 4 + el
                asrc = plsc.load_gather(rows, [rowv, 128 + hh])
                ad = plsc.load_gather(adr, [rowv, hh])
                ev = plsc.load_gather(e4v, [rowv, hh])
                al = asrc + ad + ev
                al = jnp.maximum(al, al * 0.2)
                pv = jnp.exp(al)
                plsc.store_scatter(rows, [rowv, 128 + hh], pv)
                plsc.store_scatter(rows, [rowv, 132 + hh], ev)
                for t in range(4):
                    e = j * 4 + t
                    for h in range(H):
                        ps = pv[t * 4 + h]
                        for q in range(2):
                            off = h * 32 + q * 16
                            rows[e, pl.ds(off, 16)] = (
                                rows[e, pl.ds(off, 16)] * ps)
                return 0

            lax.fori_loop(0, nch4, pstep, 0)

        def chunk(i, _):
            base = wid * EW + i * CH
            pltpu.sync_copy(src_h.at[pl.ds(base, CH)], srci)
            pltpu.sync_copy(dst_h.at[pl.ds(base, CH)], dsti)
            pltpu.sync_copy(e4_h.at[pl.ds(base, CH)], e4v)
            cp1 = pltpu.async_copy(table_h.at[srci], rows, sem1)
            cp2 = pltpu.async_copy(adst_h.at[dsti], adr, sem2)
            cp1.wait()
            cp2.wait()
            compute(CH // 4)
            pltpu.sync_copy(rows, acc.at[dsti], add=True)
            return 0

        lax.fori_loop(0, NCH, chunk, 0)

        tbase = wid * EW + NCH * CH
        pltpu.sync_copy(src_h.at[pl.ds(tbase, TAIL)], srct)
        pltpu.sync_copy(dst_h.at[pl.ds(tbase, TAIL)], dstt)
        pltpu.sync_copy(e4_h.at[pl.ds(tbase, TAIL)], e4v.at[pl.ds(0, TAIL)])
        cp1 = pltpu.async_copy(table_h.at[srct], rows.at[pl.ds(0, TAIL)], sem1)
        cp2 = pltpu.async_copy(adst_h.at[dstt], adr.at[pl.ds(0, TAIL)], sem2)
        cp1.wait()
        cp2.wait()
        compute(TAIL // 4)
        pltpu.sync_copy(rows.at[pl.ds(0, TAIL)], acc.at[dstt], add=True)

        plsc.subcore_barrier()
        doff = pl.multiple_of(sid * ZTILE, 8)

        @pl.when(sid < NS - 1)
        def _():
            pltpu.sync_copy(acc.at[pl.ds(doff, ZTILE)],
                            out_h.at[cid, pl.ds(doff, ZTILE)])

        @pl.when(sid == NS - 1)
        def _():
            pltpu.sync_copy(acc.at[pl.ds(doff, ZLAST)],
                            out_h.at[cid, pl.ds(doff, ZLAST)])

    return body(table, adst, src, dst, e4)


def _layer_norm(x, g, b, eps=1e-5):
    mu = jnp.mean(x, axis=-1, keepdims=True)
    var = jnp.mean((x - mu) ** 2, axis=-1, keepdims=True)
    return (x - mu) / jnp.sqrt(var + eps) * g + b


def _matmul_relu_kernel(x_ref, w_ref, b_ref, o_ref):
    o_ref[...] = jax.nn.relu(
        jnp.dot(x_ref[...], w_ref[...], preferred_element_type=jnp.float32)
        + b_ref[...]
    )


def _proj_relu(x, w, b):
    m = x.shape[0]
    bm = 400
    return pl.pallas_call(
        _matmul_relu_kernel,
        grid=(m // bm,),
        in_specs=[
            pl.BlockSpec((bm, x.shape[1]), lambda i: (i, 0)),
            pl.BlockSpec((x.shape[1], w.shape[1]), lambda i: (0, 0)),
            pl.BlockSpec((1, w.shape[1]), lambda i: (0, 0)),
        ],
        out_specs=pl.BlockSpec((bm, w.shape[1]), lambda i: (i, 0)),
        out_shape=jax.ShapeDtypeStruct((m, w.shape[1]), jnp.float32),
    )(x, w, b[None])


def _gat_layer(x, h, edge_attr, src, dst, e4, g):
    n = x.shape[0]
    inp = jnp.concatenate([x, h], axis=-1)
    xw = inp @ g['W']
    xwr = xw.reshape(n, H, C)
    a_src = jnp.sum(xwr * g['att_src'][None], axis=-1)
    a_dst = jnp.sum(xwr * g['att_dst'][None], axis=-1)

    table = jnp.concatenate(
        [xw, a_src, jnp.zeros((n, 4), jnp.float32),
         jnp.ones((n, 1), jnp.float32), jnp.zeros((n, 7), jnp.float32)],
        axis=1)
    adst16 = jnp.concatenate([a_dst, jnp.zeros((n, 12), jnp.float32)], axis=1)

    parts = _sc_edge_pass(table, adst16, src, dst, e4)
    s = parts[0] + parts[1]
    num = s[:, :128]
    den = s[:, 128:132]
    se4 = s[:, 132:136]
    deg = s[:, 136]

    loop_a = se4 / jnp.maximum(deg, 1.0)[:, None]
    p_self = jnp.exp(jax.nn.leaky_relu(a_src + a_dst + loop_a, 0.2))
    den_t = den + p_self + 1e-16
    num_t = num + xw * jnp.repeat(p_self, C, axis=1)
    out = num_t / jnp.repeat(den_t, C, axis=1)
    return out + g['bias']


def kernel(x, edge_attr, params, edge_index, batch):
    p = params
    src, dst = edge_index[0], edge_index[1]
    h = _proj_relu(x, p['W_in'], p['b_in'])

    e4s = []
    for l in range(2):
        g = p['gat'][l]
        A = (g['W_e'].reshape(DE, H, C) * g['att_e'][None]).sum(-1)
        e4s.append(edge_attr @ A)

    for l in range(2):
        g = p['gat'][l]
        h = _gat_layer(x, h, edge_attr, src, dst, e4s[l], g)
        h = jax.nn.elu(_layer_norm(h, g['ln_g'], g['ln_b']))

    ge = _layer_norm(jax.nn.relu(h @ p['Wg'] + p['bg']), p['lng_g'], p['lng_b'])
    cnt = jax.ops.segment_sum(jnp.ones(N, jnp.float32), batch, B)
    ge = jax.ops.segment_sum(ge, batch, B) / jnp.maximum(cnt, 1.0)[:, None]
    h = _layer_norm(jax.nn.relu(h @ p['Wo'] + p['bo']), p['lnl_g'], p['lnl_b'])
    node_in = jnp.concatenate([x, h], axis=-1)
    z = jax.nn.relu(node_in @ p['Wn1'] + p['bn1'])
    z = jax.nn.relu(z @ p['Wn2'] + p['bn2'])
    no = (z @ p['Wn3'] + p['bn3'])[:, 0]
    z = jax.nn.relu(ge @ p['Wg1'] + p['bg1'])
    z = jax.nn.relu(z @ p['Wg2'] + p['bg2'])
    go = (z @ p['Wg3'] + p['bg3'])[:, 0]
    return jnp.concatenate([no, go], axis=-1)
